# Initial kernel scaffold; baseline (speedup 1.0000x reference)
import jax
import jax.numpy as jnp
from jax.experimental import pallas as pl


def kernel(edge_index, adj_values, uEmbeds, iEmbeds):
    raise NotImplementedError("write your pallas kernel here")


# speedup vs baseline: 4.4282x; 4.4282x over previous
import functools

import jax
import jax.numpy as jnp
from jax import lax
from jax.experimental import pallas as pl
from jax.experimental.pallas import tpu as pltpu
from jax.experimental.pallas import tpu_sc as plsc

_USER = 5000
_ITEM = 5000
_LATDIM = 128
_EPS = 0.1
_N = _USER + _ITEM
_E = 320000
_NCORE = 2
_NSUB = 16
_NW = _NCORE * _NSUB
_BLK = 128
_NBLK_TOT = _E // _BLK
_BLK_PER_W = _NBLK_TOT // _NW
_BLK_EXTRA = _NBLK_TOT - _BLK_PER_W * _NW
_ROWS_SUB = 632
_ROWS_LAST = _N - 15 * _ROWS_SUB


def _spmm_body(tbl_hbm, dst_hbm, src_hbm, val_hbm, zeros_hbm, out_hbm,
               acc, dst_v, src_v, val_v, rows_v, sem):
    c = lax.axis_index("c")
    s = lax.axis_index("s")
    wid = s * _NCORE + c

    @pl.when(s < 15)
    def _():
        pltpu.sync_copy(zeros_hbm, acc.at[pl.ds(s * _ROWS_SUB, _ROWS_SUB)])

    @pl.when(s == 15)
    def _():
        pltpu.sync_copy(zeros_hbm.at[pl.ds(0, _ROWS_LAST)],
                        acc.at[pl.ds(15 * _ROWS_SUB, _ROWS_LAST)])

    plsc.subcore_barrier()

    nblk = _BLK_PER_W + jnp.where(wid < _BLK_EXTRA, 1, 0)
    blk0 = wid * _BLK_PER_W + jnp.minimum(wid, _BLK_EXTRA)

    def block(b, _):
        off = (blk0 + b) * _BLK
        pltpu.sync_copy(dst_hbm.at[pl.ds(off, _BLK)], dst_v.at[0])
        pltpu.sync_copy(src_hbm.at[pl.ds(off, _BLK)], src_v.at[0])
        pltpu.sync_copy(val_hbm.at[pl.ds(off, _BLK)], val_v)
        pltpu.async_copy(tbl_hbm.at[src_v.at[0]], rows_v, sem).wait()

        for g in range(_BLK // 16):
            vv = val_v[pl.ds(g * 16, 16)]
            for l in range(16):
                v16 = vv.at[jnp.full((16,), l, jnp.int32)].get(
                    mode="promise_in_bounds")
                e = g * 16 + l
                for k in range(_LATDIM // 16):
                    rows_v[e, pl.ds(k * 16, 16)] = (
                        rows_v[e, pl.ds(k * 16, 16)] * v16)

        pltpu.sync_copy(rows_v, acc.at[dst_v.at[0]], add=True)
        return _

    lax.fori_loop(0, nblk, block, None)
    plsc.subcore_barrier()

    coff = c * _N

    @pl.when(s < 15)
    def _():
        pltpu.sync_copy(acc.at[pl.ds(s * _ROWS_SUB, _ROWS_SUB)],
                        out_hbm.at[pl.ds(coff + s * _ROWS_SUB, _ROWS_SUB)])

    @pl.when(s == 15)
    def _():
        pltpu.sync_copy(acc.at[pl.ds(15 * _ROWS_SUB, _ROWS_LAST)],
                        out_hbm.at[pl.ds(coff + 15 * _ROWS_SUB, _ROWS_LAST)])


@functools.partial(
    pl.kernel,
    out_type=jax.ShapeDtypeStruct((2 * _N, _LATDIM), jnp.float32),
    mesh=plsc.VectorSubcoreMesh(core_axis_name="c", subcore_axis_name="s"),
    scratch_types=[
        pltpu.VMEM_SHARED((_N, _LATDIM), jnp.float32),
        pltpu.VMEM((1, _BLK), jnp.int32),
        pltpu.VMEM((1, _BLK), jnp.int32),
        pltpu.VMEM((_BLK,), jnp.float32),
        pltpu.VMEM((_BLK, _LATDIM), jnp.float32),
        pltpu.SemaphoreType.DMA,
    ],
)
def _spmm_pass(tbl_hbm, dst_hbm, src_hbm, val_hbm, zeros_hbm, out_hbm,
               acc, dst_v, src_v, val_v, rows_v, sem):
    _spmm_body(tbl_hbm, dst_hbm, src_hbm, val_hbm, zeros_hbm, out_hbm,
               acc, dst_v, src_v, val_v, rows_v, sem)


def _l2n(x):
    nrm = jnp.linalg.norm(x, axis=-1, keepdims=True)
    return x / jnp.maximum(nrm, 1e-12)


def kernel(edge_index, adj_values, uEmbeds, iEmbeds):
    ini = jnp.concatenate([uEmbeds, iEmbeds], axis=0)
    dst = edge_index[0]
    src = edge_index[1]
    zeros = jnp.zeros((_ROWS_SUB, _LATDIM), jnp.float32)

    def spmm(t):
        parts = _spmm_pass(t, dst, src, adj_values, zeros)
        return parts[:_N] + parts[_N:]

    e1 = spmm(ini)

    n10 = _l2n(jax.random.uniform(jax.random.fold_in(jax.random.key(42), 2),
                                  (_N, _LATDIM), dtype=jnp.float32))
    n11 = _l2n(jax.random.uniform(jax.random.fold_in(jax.random.key(42), 3),
                                  (_N, _LATDIM), dtype=jnp.float32))
    n20 = _l2n(jax.random.uniform(jax.random.fold_in(jax.random.key(42), 4),
                                  (_N, _LATDIM), dtype=jnp.float32))
    n21 = _l2n(jax.random.uniform(jax.random.fold_in(jax.random.key(42), 5),
                                  (_N, _LATDIM), dtype=jnp.float32))

    p11 = e1 + jnp.sign(e1) * n10 * _EPS
    p12 = e1 + jnp.sign(e1) * n20 * _EPS

    e2 = spmm(e1)
    q1 = spmm(p11)
    q2 = spmm(p12)

    mainE = (e1 + e2) * 0.5
    p21 = q1 + jnp.sign(q1) * n11 * _EPS
    p22 = q2 + jnp.sign(q2) * n21 * _EPS
    pert1 = (p11 + p21) * 0.5
    pert2 = (p12 + p22) * 0.5

    return (mainE[:_USER], mainE[_USER:],
            pert1[:_USER], pert1[_USER:],
            pert2[:_USER], pert2[_USER:])
